# Initial kernel scaffold; baseline (speedup 1.0000x reference)
import functools

import jax
import jax.numpy as jnp
from jax import lax
from jax.experimental import pallas as pl
from jax.experimental.pallas import tpu as pltpu
from jax.experimental.pallas import tpu_sc as plsc

N_NODES = 10000
N_EDGES = 160000
D_FEAT = 256
D_HALF = 128

N_CORES = 2
N_SUBCORES = 16
N_WORKERS = N_CORES * N_SUBCORES

N_PAD = 10240
ROWS_PER_SUB = N_PAD // N_SUBCORES
E_PAD = 163840
E_PER_W = E_PAD // N_WORKERS
CHUNK = 128
CHUNKS_PER_W = E_PER_W // CHUNK
DUMMY_DST = N_NODES


def _sc_aggregate(x2, src, dst):
    mesh = plsc.VectorSubcoreMesh(
        core_axis_name="c", subcore_axis_name="s",
        num_cores=N_CORES, num_subcores=N_SUBCORES)

    @functools.partial(
        pl.kernel,
        out_type=(
            jax.ShapeDtypeStruct((N_CORES, N_PAD, D_HALF), jnp.float32),
            jax.ShapeDtypeStruct((N_CORES, N_PAD, 16), jnp.float32),
        ),
        mesh=mesh,
        scratch_types=[
            pltpu.VMEM((CHUNK,), jnp.int32),
            pltpu.VMEM((CHUNK,), jnp.int32),
            pltpu.VMEM((CHUNK,), jnp.int32),
            pltpu.VMEM((CHUNK, D_HALF), jnp.float32),
            pltpu.VMEM((CHUNK, 16), jnp.float32),
            pltpu.VMEM_SHARED((N_PAD, D_HALF), jnp.float32),
            pltpu.VMEM_SHARED((N_PAD, 16), jnp.float32),
            pltpu.SemaphoreType.DMA,
        ],
    )
    def sc_kernel(x2_hbm, src_hbm, dst_hbm, agg_hbm, deg_hbm,
                  src_v, dst_v, gidx_v, rows_v, ones_v, agg_sh, deg_sh, sem):
        c = lax.axis_index("c")
        s = lax.axis_index("s")

        @pl.loop(0, CHUNK)
        def _(i):
            ones_v[i, :] = jnp.zeros((16,), jnp.float32)

            @pl.loop(0, D_HALF, step=16)
            def _(j):
                rows_v[i, pl.ds(j, 16)] = jnp.zeros((16,), jnp.float32)

        @pl.loop(0, ROWS_PER_SUB // CHUNK)
        def _(t):
            r0 = s * ROWS_PER_SUB + t * CHUNK
            pltpu.sync_copy(rows_v, agg_sh.at[pl.ds(r0, CHUNK)])
            pltpu.sync_copy(ones_v, deg_sh.at[pl.ds(r0, CHUNK)])

        @pl.loop(0, CHUNK)
        def _(i):
            ones_v[i, :] = jnp.full((16,), 1.0, jnp.float32)

        plsc.subcore_barrier()

        wid = s * N_CORES + c
        base0 = wid * E_PER_W

        @pl.loop(0, CHUNKS_PER_W)
        def _(j):
            base = base0 + j * CHUNK
            pltpu.sync_copy(src_hbm.at[pl.ds(base, CHUNK)], src_v)
            pltpu.sync_copy(dst_hbm.at[pl.ds(base, CHUNK)], dst_v)

            @pl.loop(0, CHUNK, step=16)
            def _(t):
                gidx_v[pl.ds(t, 16)] = src_v[pl.ds(t, 16)] * 2 + c

            pltpu.async_copy(x2_hbm.at[gidx_v], rows_v, sem).wait()
            pltpu.sync_copy(rows_v, agg_sh.at[dst_v], add=True)

            @pl.when(j % 2 == c)
            def _():
                pltpu.sync_copy(ones_v, deg_sh.at[dst_v], add=True)

        plsc.subcore_barrier()

        r0 = s * ROWS_PER_SUB
        pltpu.sync_copy(agg_sh.at[pl.ds(r0, ROWS_PER_SUB)],
                        agg_hbm.at[c, pl.ds(r0, ROWS_PER_SUB)])
        pltpu.sync_copy(deg_sh.at[pl.ds(r0, ROWS_PER_SUB)],
                        deg_hbm.at[c, pl.ds(r0, ROWS_PER_SUB)])

    return sc_kernel(x2, src, dst)


BLOCK_M = 400


def _tc_head(x, agg, deg, W_self, W_neigh, bs2, W1, b12, W2, b22):
    grid = (N_NODES // BLOCK_M,)

    def body(x_ref, agg_ref, deg_ref, ws_ref, wn_ref, bs_ref,
             w1_ref, b1_ref, w2_ref, b2_ref, o_ref):
        deg_sum = deg_ref[0] + deg_ref[1]
        inv = 1.0 / jnp.maximum(deg_sum[:, :1], 1.0)
        a0 = agg_ref[0] * inv
        a1 = agg_ref[1] * inv
        h = (jnp.dot(x_ref[...], ws_ref[...], preferred_element_type=jnp.float32)
             + jnp.dot(a0, wn_ref[:D_HALF, :], preferred_element_type=jnp.float32)
             + jnp.dot(a1, wn_ref[D_HALF:, :], preferred_element_type=jnp.float32)
             + bs_ref[...])
        h = jnp.maximum(h, 0.0)
        h1 = jnp.maximum(
            jnp.dot(h, w1_ref[...], preferred_element_type=jnp.float32)
            + b1_ref[...], 0.0)
        o_ref[...] = (jnp.dot(h1, w2_ref[...], preferred_element_type=jnp.float32)
                      + b2_ref[...])

    return pl.pallas_call(
        body,
        grid=grid,
        in_specs=[
            pl.BlockSpec((BLOCK_M, D_FEAT), lambda i: (i, 0)),
            pl.BlockSpec((N_CORES, BLOCK_M, D_HALF), lambda i: (0, i, 0)),
            pl.BlockSpec((N_CORES, BLOCK_M, 16), lambda i: (0, i, 0)),
            pl.BlockSpec((D_FEAT, D_FEAT), lambda i: (0, 0)),
            pl.BlockSpec((D_FEAT, D_FEAT), lambda i: (0, 0)),
            pl.BlockSpec((1, D_FEAT), lambda i: (0, 0)),
            pl.BlockSpec((D_FEAT, 1024), lambda i: (0, 0)),
            pl.BlockSpec((1, 1024), lambda i: (0, 0)),
            pl.BlockSpec((1024, D_FEAT), lambda i: (0, 0)),
            pl.BlockSpec((1, D_FEAT), lambda i: (0, 0)),
        ],
        out_specs=pl.BlockSpec((BLOCK_M, D_FEAT), lambda i: (i, 0)),
        out_shape=jax.ShapeDtypeStruct((N_NODES, D_FEAT), jnp.float32),
        compiler_params=pltpu.CompilerParams(
            dimension_semantics=("parallel",)),
    )(x, agg, deg, W_self, W_neigh, bs2, W1, b12, W2, b22)


def kernel(x, edge_index, W_self, W_neigh, b_sage, W1, b1, W2, b2):
    src = edge_index[0].astype(jnp.int32)
    dst = edge_index[1].astype(jnp.int32)
    pad = E_PAD - N_EDGES
    src = jnp.concatenate([src, jnp.zeros((pad,), jnp.int32)])
    dst = jnp.concatenate([dst, jnp.full((pad,), DUMMY_DST, jnp.int32)])
    x2 = x.reshape(2 * N_NODES, D_HALF)

    agg, deg = _sc_aggregate(x2, src, dst)

    return _tc_head(x, agg, deg, W_self, W_neigh,
                    b_sage.reshape(1, D_FEAT), W1, b1.reshape(1, 1024),
                    W2, b2.reshape(1, D_FEAT))


# speedup vs baseline: 2.9406x; 2.9406x over previous
import dataclasses
import functools

import jax
import jax.numpy as jnp
from jax import lax
from jax.experimental import pallas as pl
from jax.experimental.pallas import tpu as pltpu
from jax.experimental.pallas import tpu_sc as plsc

N_NODES = 10000
N_EDGES = 160000
D_FEAT = 256
D_HALF = 128
D_HID = 1024

N_CORES = 2
N_SUBCORES = 16

N_PAD = 10240
ROWS_PER_SUB = N_PAD // N_SUBCORES
DEG_PAD = 10400
E_PAD = 163840
E_PER_S = E_PAD // N_SUBCORES
CHUNK = 128
CHUNKS_PER_S = E_PER_S // CHUNK
HALF_CHUNKS = CHUNKS_PER_S // 2
DUMMY_DST = N_NODES


def _sc_aggregate(x2, src, dst):
    mesh = plsc.VectorSubcoreMesh(
        core_axis_name="c", subcore_axis_name="s",
        num_cores=N_CORES, num_subcores=N_SUBCORES)

    cp = pltpu.CompilerParams()
    if "needs_layout_passes" in pltpu.CompilerParams.__dataclass_fields__:
        cp = dataclasses.replace(cp, needs_layout_passes=False)

    @functools.partial(
        pl.kernel,
        out_type=(
            jax.ShapeDtypeStruct((N_CORES, N_PAD, D_HALF), jnp.float32),
            jax.ShapeDtypeStruct((N_CORES, N_SUBCORES, DEG_PAD), jnp.float32),
        ),
        mesh=mesh,
        compiler_params=cp,
        scratch_types=[
            pltpu.VMEM((CHUNK,), jnp.int32),
            pltpu.VMEM((CHUNK,), jnp.int32),
            pltpu.VMEM((CHUNK,), jnp.int32),
            pltpu.VMEM((CHUNK, D_HALF), jnp.float32),
            pltpu.VMEM((DEG_PAD,), jnp.float32),
            pltpu.VMEM_SHARED((N_PAD, D_HALF), jnp.float32),
            pltpu.SemaphoreType.DMA,
        ],
    )
    def sc_kernel(x2_hbm, src_hbm, dst_hbm, agg_hbm, deg_hbm,
                  src_v, dst_v, gidx_v, rows_v, deg_v, agg_sh, sem):
        c = lax.axis_index("c")
        s = lax.axis_index("s")

        @pl.loop(0, CHUNK)
        def _(i):
            @pl.loop(0, D_HALF, step=16)
            def _(j):
                rows_v[i, pl.ds(j, 16)] = jnp.zeros((16,), jnp.float32)

        @pl.loop(0, ROWS_PER_SUB // CHUNK)
        def _(t):
            pltpu.sync_copy(rows_v,
                            agg_sh.at[pl.ds(s * ROWS_PER_SUB + t * CHUNK, CHUNK)])

        @pl.loop(0, DEG_PAD, step=16)
        def _(i):
            deg_v[pl.ds(i, 16)] = jnp.zeros((16,), jnp.float32)

        plsc.subcore_barrier()

        base0 = s * E_PER_S
        ones16 = jnp.full((16,), 1.0, jnp.float32)

        def process(jj, count_deg):
            base = base0 + jj * CHUNK
            pltpu.sync_copy(src_hbm.at[pl.ds(base, CHUNK)], src_v)
            pltpu.sync_copy(dst_hbm.at[pl.ds(base, CHUNK)], dst_v)

            @pl.loop(0, CHUNK, step=16)
            def _(t):
                gidx_v[pl.ds(t, 16)] = src_v[pl.ds(t, 16)] * 2 + c

            pltpu.async_copy(x2_hbm.at[gidx_v], rows_v, sem).wait()
            pltpu.sync_copy(rows_v, agg_sh.at[dst_v], add=True)
            if count_deg:
                @pl.loop(0, CHUNK, step=16)
                def _(t):
                    plsc.addupdate_scatter(deg_v, [dst_v[pl.ds(t, 16)]], ones16)

        @pl.loop(0, HALF_CHUNKS)
        def _(j):
            process(c * HALF_CHUNKS + j, True)

        @pl.loop(0, HALF_CHUNKS)
        def _(j):
            process((1 - c) * HALF_CHUNKS + j, False)

        plsc.subcore_barrier()

        pltpu.sync_copy(agg_sh.at[pl.ds(s * ROWS_PER_SUB, ROWS_PER_SUB)],
                        agg_hbm.at[c, pl.ds(s * ROWS_PER_SUB, ROWS_PER_SUB)])
        pltpu.sync_copy(deg_v, deg_hbm.at[c, s])

    return sc_kernel(x2, src, dst)


BLOCK_M = 400
GRID_M = N_NODES // BLOCK_M


def _tc_head(x, agg, deg4, W_self, W_neigh, bs2, W1, b12, W2, b22):
    def body(x_ref, agg_ref, deg_ref, ws_ref, wn_ref, bs_ref,
             w1_ref, b1_ref, w2_ref, b2_ref, o_ref):
        i = pl.program_id(0)
        dblk = deg_ref[:, :, pl.ds(i, 1), :]
        deg_sum = jnp.sum(dblk, axis=(0, 1, 2))
        inv_row = 1.0 / jnp.maximum(deg_sum, 1.0)
        inv = jnp.transpose(inv_row.reshape(1, BLOCK_M))
        a0 = agg_ref[0] * inv
        a1 = agg_ref[1] * inv
        h = (jnp.dot(x_ref[...], ws_ref[...], preferred_element_type=jnp.float32)
             + jnp.dot(a0, wn_ref[:D_HALF, :], preferred_element_type=jnp.float32)
             + jnp.dot(a1, wn_ref[D_HALF:, :], preferred_element_type=jnp.float32)
             + bs_ref[...])
        h = jnp.maximum(h, 0.0)
        h1 = jnp.maximum(
            jnp.dot(h, w1_ref[...], preferred_element_type=jnp.float32)
            + b1_ref[...], 0.0)
        o_ref[...] = (jnp.dot(h1, w2_ref[...], preferred_element_type=jnp.float32)
                      + b2_ref[...])

    return pl.pallas_call(
        body,
        grid=(GRID_M,),
        in_specs=[
            pl.BlockSpec((BLOCK_M, D_FEAT), lambda i: (i, 0)),
            pl.BlockSpec((N_CORES, BLOCK_M, D_HALF), lambda i: (0, i, 0)),
            pl.BlockSpec((N_CORES, N_SUBCORES, DEG_PAD // BLOCK_M, BLOCK_M),
                         lambda i: (0, 0, 0, 0)),
            pl.BlockSpec((D_FEAT, D_FEAT), lambda i: (0, 0)),
            pl.BlockSpec((D_FEAT, D_FEAT), lambda i: (0, 0)),
            pl.BlockSpec((1, D_FEAT), lambda i: (0, 0)),
            pl.BlockSpec((D_FEAT, D_HID), lambda i: (0, 0)),
            pl.BlockSpec((1, D_HID), lambda i: (0, 0)),
            pl.BlockSpec((D_HID, D_FEAT), lambda i: (0, 0)),
            pl.BlockSpec((1, D_FEAT), lambda i: (0, 0)),
        ],
        out_specs=pl.BlockSpec((BLOCK_M, D_FEAT), lambda i: (i, 0)),
        out_shape=jax.ShapeDtypeStruct((N_NODES, D_FEAT), jnp.float32),
        compiler_params=pltpu.CompilerParams(
            dimension_semantics=("parallel",)),
    )(x, agg, deg4, W_self, W_neigh, bs2, W1, b12, W2, b22)


def kernel(x, edge_index, W_self, W_neigh, b_sage, W1, b1, W2, b2):
    src = edge_index[0].astype(jnp.int32)
    dst = edge_index[1].astype(jnp.int32)
    pad = E_PAD - N_EDGES
    src = jnp.concatenate([src, jnp.zeros((pad,), jnp.int32)])
    dst = jnp.concatenate([dst, jnp.full((pad,), DUMMY_DST, jnp.int32)])
    x2 = x.reshape(2 * N_NODES, D_HALF)

    agg, deg = _sc_aggregate(x2, src, dst)
    deg4 = deg.reshape(N_CORES, N_SUBCORES, DEG_PAD // BLOCK_M, BLOCK_M)

    return _tc_head(x, agg, deg4, W_self, W_neigh,
                    b_sage.reshape(1, D_FEAT), W1, b1.reshape(1, D_HID),
                    W2, b2.reshape(1, D_FEAT))


# speedup vs baseline: 3.6175x; 1.2302x over previous
import dataclasses
import functools

import jax
import jax.numpy as jnp
from jax import lax
from jax.experimental import pallas as pl
from jax.experimental.pallas import tpu as pltpu
from jax.experimental.pallas import tpu_sc as plsc

N_NODES = 10000
N_EDGES = 160000
D_FEAT = 256
D_HALF = 128
D_HID = 1024

N_CORES = 2
N_SUBCORES = 16

N_PAD = 10240
ROWS_PER_SUB = N_PAD // N_SUBCORES
DEG_PAD = 10400
E_PAD = 163840
E_PER_S = E_PAD // N_SUBCORES
CHUNK = 64
NCH = E_PER_S // CHUNK
NPAIR = NCH // 2
W_CHUNKS = E_PAD // (32 * CHUNK)
DUMMY_DST = N_NODES
PK_SHIFT = 14
PK_MASK = (1 << PK_SHIFT) - 1


def _sc_compiler_params():
    cp = pltpu.CompilerParams()
    if "needs_layout_passes" in pltpu.CompilerParams.__dataclass_fields__:
        cp = dataclasses.replace(cp, needs_layout_passes=False)
    return cp


_MESH = plsc.VectorSubcoreMesh(
    core_axis_name="c", subcore_axis_name="s",
    num_cores=N_CORES, num_subcores=N_SUBCORES)


def _sc_aggregate(x2, packed):
    @functools.partial(
        pl.kernel,
        out_type=jax.ShapeDtypeStruct((N_CORES, N_PAD, D_HALF), jnp.float32),
        mesh=_MESH,
        compiler_params=_sc_compiler_params(),
        scratch_types=[
            pltpu.VMEM((2, 2, CHUNK), jnp.int32),
            pltpu.VMEM((4, CHUNK), jnp.int32),
            pltpu.VMEM((4, CHUNK), jnp.int32),
            pltpu.VMEM((CHUNK, D_HALF), jnp.float32),
            pltpu.VMEM((CHUNK, D_HALF), jnp.float32),
            pltpu.VMEM_SHARED((N_PAD, D_HALF), jnp.float32),
            pltpu.SemaphoreType.DMA,
            pltpu.SemaphoreType.DMA,
            pltpu.SemaphoreType.DMA,
            pltpu.SemaphoreType.DMA,
            pltpu.SemaphoreType.DMA,
            pltpu.SemaphoreType.DMA,
        ],
    )
    def sc_kernel(x2_hbm, pk_hbm, agg_hbm,
                  pb, gb, db, rows0_v, rows1_v, agg_sh,
                  g0, g1, s0, s1, i0, i1):
        c = lax.axis_index("c")
        s = lax.axis_index("s")
        rows = (rows0_v, rows1_v)
        gsem = (g0, g1)
        ssem = (s0, s1)
        isem = (i0, i1)
        base = s * NCH

        def unpack(q):
            for kk in range(2):
                for t in range(0, CHUNK, 16):
                    v = pb[q, kk, pl.ds(t, 16)]
                    gb[2 * q + kk, pl.ds(t, 16)] = (v >> PK_SHIFT) * 2 + c
                    db[2 * q + kk, pl.ds(t, 16)] = v & PK_MASK

        @pl.loop(0, CHUNK)
        def _(i):
            @pl.loop(0, D_HALF, step=16)
            def _(j):
                rows0_v[i, pl.ds(j, 16)] = jnp.zeros((16,), jnp.float32)

        @pl.loop(0, ROWS_PER_SUB // CHUNK)
        def _(t):
            pltpu.sync_copy(
                rows0_v, agg_sh.at[pl.ds(s * ROWS_PER_SUB + t * CHUNK, CHUNK)])

        pltpu.sync_copy(pk_hbm.at[pl.ds(base, 2)], pb.at[0])
        unpack(0)
        pltpu.async_copy(pk_hbm.at[pl.ds(base + 2, 2)], pb.at[1], i1)
        pltpu.async_copy(x2_hbm.at[gb.at[0]], rows0_v, g0)
        pltpu.async_copy(x2_hbm.at[gb.at[1]], rows1_v, g1)
        plsc.subcore_barrier()

        def body(p, q, pf_idx, pf_gather):
            qn = 1 - q
            if pf_idx:
                pltpu.async_copy(pk_hbm.at[pl.ds(base + 2 * p + 4, 2)],
                                 pb.at[q], isem[q])
            for kk in range(2):
                pltpu.make_async_copy(x2_hbm.at[gb.at[2 * q + kk]],
                                      rows[kk], gsem[kk]).wait()
                pltpu.async_copy(rows[kk], agg_sh.at[db.at[2 * q + kk]],
                                 ssem[kk], add=True)
            if pf_gather:
                pltpu.make_async_copy(pk_hbm.at[pl.ds(base, 2)],
                                      pb.at[qn], isem[qn]).wait()
                unpack(qn)
                for kk in range(2):
                    pltpu.make_async_copy(rows[kk],
                                          agg_sh.at[db.at[2 * q + kk]],
                                          ssem[kk]).wait()
                    pltpu.async_copy(x2_hbm.at[gb.at[2 * qn + kk]],
                                     rows[kk], gsem[kk])

        @pl.loop(0, NPAIR // 2 - 1)
        def _(p2):
            body(2 * p2, 0, True, True)
            body(2 * p2 + 1, 1, True, True)

        body(NPAIR - 2, 0, False, True)
        body(NPAIR - 1, 1, False, False)
        for kk in range(2):
            pltpu.make_async_copy(rows[kk], agg_sh.at[db.at[2 + kk]],
                                  ssem[kk]).wait()

        plsc.subcore_barrier()

        pltpu.sync_copy(agg_sh.at[pl.ds(s * ROWS_PER_SUB, ROWS_PER_SUB)],
                        agg_hbm.at[c, pl.ds(s * ROWS_PER_SUB, ROWS_PER_SUB)])

    return sc_kernel(x2, packed)


def _sc_degrees(packed):
    @functools.partial(
        pl.kernel,
        out_type=jax.ShapeDtypeStruct((N_CORES, N_SUBCORES, DEG_PAD),
                                      jnp.float32),
        mesh=_MESH,
        compiler_params=_sc_compiler_params(),
        scratch_types=[
            pltpu.VMEM((W_CHUNKS, CHUNK), jnp.int32),
            pltpu.VMEM((DEG_PAD,), jnp.float32),
        ],
    )
    def deg_kernel(pk_hbm, deg_hbm, slab_v, deg_v):
        c = lax.axis_index("c")
        s = lax.axis_index("s")
        w = s * N_CORES + c

        pltpu.sync_copy(pk_hbm.at[pl.ds(w * W_CHUNKS, W_CHUNKS)], slab_v)

        @pl.loop(0, DEG_PAD, step=16)
        def _(i):
            deg_v[pl.ds(i, 16)] = jnp.zeros((16,), jnp.float32)

        ones16 = jnp.full((16,), 1.0, jnp.float32)

        @pl.loop(0, W_CHUNKS)
        def _(i):
            @pl.loop(0, CHUNK, step=16)
            def _(t):
                d = slab_v[i, pl.ds(t, 16)] & PK_MASK
                plsc.addupdate_scatter(deg_v, [d], ones16)

        pltpu.sync_copy(deg_v, deg_hbm.at[c, s])

    return deg_kernel(packed)


BLOCK_M = 400
GRID_M = N_NODES // BLOCK_M


def _tc_head(x, agg, deg4, W_self, W_neigh, bs2, W1, b12, W2, b22):
    def body(x_ref, agg_ref, deg_ref, ws_ref, wn_ref, bs_ref,
             w1_ref, b1_ref, w2_ref, b2_ref, o_ref):
        i = pl.program_id(0)
        dblk = deg_ref[:, :, pl.ds(i, 1), :]
        deg_sum = jnp.sum(dblk, axis=(0, 1, 2))
        inv_row = 1.0 / jnp.maximum(deg_sum, 1.0)
        inv = jnp.transpose(inv_row.reshape(1, BLOCK_M))
        a0 = agg_ref[0] * inv
        a1 = agg_ref[1] * inv
        h = (jnp.dot(x_ref[...], ws_ref[...], preferred_element_type=jnp.float32)
             + jnp.dot(a0, wn_ref[:D_HALF, :], preferred_element_type=jnp.float32)
             + jnp.dot(a1, wn_ref[D_HALF:, :], preferred_element_type=jnp.float32)
             + bs_ref[...])
        h = jnp.maximum(h, 0.0)
        h1 = jnp.maximum(
            jnp.dot(h, w1_ref[...], preferred_element_type=jnp.float32)
            + b1_ref[...], 0.0)
        o_ref[...] = (jnp.dot(h1, w2_ref[...], preferred_element_type=jnp.float32)
                      + b2_ref[...])

    return pl.pallas_call(
        body,
        grid=(GRID_M,),
        in_specs=[
            pl.BlockSpec((BLOCK_M, D_FEAT), lambda i: (i, 0)),
            pl.BlockSpec((N_CORES, BLOCK_M, D_HALF), lambda i: (0, i, 0)),
            pl.BlockSpec((N_CORES, N_SUBCORES, DEG_PAD // BLOCK_M, BLOCK_M),
                         lambda i: (0, 0, 0, 0)),
            pl.BlockSpec((D_FEAT, D_FEAT), lambda i: (0, 0)),
            pl.BlockSpec((D_FEAT, D_FEAT), lambda i: (0, 0)),
            pl.BlockSpec((1, D_FEAT), lambda i: (0, 0)),
            pl.BlockSpec((D_FEAT, D_HID), lambda i: (0, 0)),
            pl.BlockSpec((1, D_HID), lambda i: (0, 0)),
            pl.BlockSpec((D_HID, D_FEAT), lambda i: (0, 0)),
            pl.BlockSpec((1, D_FEAT), lambda i: (0, 0)),
        ],
        out_specs=pl.BlockSpec((BLOCK_M, D_FEAT), lambda i: (i, 0)),
        out_shape=jax.ShapeDtypeStruct((N_NODES, D_FEAT), jnp.float32),
        compiler_params=pltpu.CompilerParams(
            dimension_semantics=("parallel",)),
    )(x, agg, deg4, W_self, W_neigh, bs2, W1, b12, W2, b22)


def kernel(x, edge_index, W_self, W_neigh, b_sage, W1, b1, W2, b2):
    src = edge_index[0].astype(jnp.int32)
    dst = edge_index[1].astype(jnp.int32)
    pad = E_PAD - N_EDGES
    src = jnp.concatenate([src, jnp.zeros((pad,), jnp.int32)])
    dst = jnp.concatenate([dst, jnp.full((pad,), DUMMY_DST, jnp.int32)])
    packed = ((src << PK_SHIFT) | dst).reshape(E_PAD // CHUNK, CHUNK)
    x2 = x.reshape(2 * N_NODES, D_HALF)

    agg = _sc_aggregate(x2, packed)
    deg = _sc_degrees(packed)
    deg4 = deg.reshape(N_CORES, N_SUBCORES, DEG_PAD // BLOCK_M, BLOCK_M)

    return _tc_head(x, agg, deg4, W_self, W_neigh,
                    b_sage.reshape(1, D_FEAT), W1, b1.reshape(1, D_HID),
                    W2, b2.reshape(1, D_FEAT))


# speedup vs baseline: 3.7492x; 1.0364x over previous
import dataclasses
import functools

import jax
import jax.numpy as jnp
from jax import lax
from jax.experimental import pallas as pl
from jax.experimental.pallas import tpu as pltpu
from jax.experimental.pallas import tpu_sc as plsc

N_NODES = 10000
N_EDGES = 160000
D_FEAT = 256
D_HALF = 128
D_HID = 1024

N_CORES = 2
N_SUBCORES = 16

N_PAD = 10240
ROWS_PER_SUB = N_PAD // N_SUBCORES
DEG_PAD = 10400
E_PAD = 163840
E_PER_S = E_PAD // N_SUBCORES
CHUNK = 80
NCH = E_PER_S // CHUNK
NPAIR = NCH // 2
W_CHUNKS = E_PAD // (32 * CHUNK)
DUMMY_DST = N_NODES
PK_SHIFT = 14
PK_MASK = (1 << PK_SHIFT) - 1


def _sc_compiler_params():
    cp = pltpu.CompilerParams()
    if "needs_layout_passes" in pltpu.CompilerParams.__dataclass_fields__:
        cp = dataclasses.replace(cp, needs_layout_passes=False)
    return cp


_MESH = plsc.VectorSubcoreMesh(
    core_axis_name="c", subcore_axis_name="s",
    num_cores=N_CORES, num_subcores=N_SUBCORES)


def _sc_aggregate(x2, packed):
    @functools.partial(
        pl.kernel,
        out_type=jax.ShapeDtypeStruct((N_CORES, N_PAD, D_HALF), jnp.float32),
        mesh=_MESH,
        compiler_params=_sc_compiler_params(),
        scratch_types=[
            pltpu.VMEM((2, 2, CHUNK), jnp.int32),
            pltpu.VMEM((4, CHUNK), jnp.int32),
            pltpu.VMEM((4, CHUNK), jnp.int32),
            pltpu.VMEM((CHUNK, D_HALF), jnp.float32),
            pltpu.VMEM((CHUNK, D_HALF), jnp.float32),
            pltpu.VMEM_SHARED((N_PAD, D_HALF), jnp.float32),
            pltpu.SemaphoreType.DMA,
            pltpu.SemaphoreType.DMA,
            pltpu.SemaphoreType.DMA,
            pltpu.SemaphoreType.DMA,
            pltpu.SemaphoreType.DMA,
            pltpu.SemaphoreType.DMA,
        ],
    )
    def sc_kernel(x2_hbm, pk_hbm, agg_hbm,
                  pb, gb, db, rows0_v, rows1_v, agg_sh,
                  g0, g1, s0, s1, i0, i1):
        c = lax.axis_index("c")
        s = lax.axis_index("s")
        rows = (rows0_v, rows1_v)
        gsem = (g0, g1)
        ssem = (s0, s1)
        isem = (i0, i1)
        base = s * NCH

        def unpack(q):
            for kk in range(2):
                for t in range(0, CHUNK, 16):
                    v = pb[q, kk, pl.ds(t, 16)]
                    gb[2 * q + kk, pl.ds(t, 16)] = (v >> PK_SHIFT) * 2 + c
                    db[2 * q + kk, pl.ds(t, 16)] = v & PK_MASK

        @pl.loop(0, CHUNK)
        def _(i):
            @pl.loop(0, D_HALF, step=16)
            def _(j):
                rows0_v[i, pl.ds(j, 16)] = jnp.zeros((16,), jnp.float32)

        @pl.loop(0, ROWS_PER_SUB // CHUNK)
        def _(t):
            pltpu.sync_copy(
                rows0_v, agg_sh.at[pl.ds(s * ROWS_PER_SUB + t * CHUNK, CHUNK)])

        pltpu.sync_copy(pk_hbm.at[pl.ds(base, 2)], pb.at[0])
        unpack(0)
        pltpu.async_copy(pk_hbm.at[pl.ds(base + 2, 2)], pb.at[1], i1)
        pltpu.async_copy(x2_hbm.at[gb.at[0]], rows0_v, g0)
        pltpu.async_copy(x2_hbm.at[gb.at[1]], rows1_v, g1)
        plsc.subcore_barrier()

        def body(p, q, pf_idx, pf_gather):
            qn = 1 - q
            if pf_idx:
                pltpu.async_copy(pk_hbm.at[pl.ds(base + 2 * p + 4, 2)],
                                 pb.at[q], isem[q])
            for kk in range(2):
                pltpu.make_async_copy(x2_hbm.at[gb.at[2 * q + kk]],
                                      rows[kk], gsem[kk]).wait()
                pltpu.async_copy(rows[kk], agg_sh.at[db.at[2 * q + kk]],
                                 ssem[kk], add=True)
            if pf_gather:
                pltpu.make_async_copy(pk_hbm.at[pl.ds(base, 2)],
                                      pb.at[qn], isem[qn]).wait()
                unpack(qn)
                for kk in range(2):
                    pltpu.make_async_copy(rows[kk],
                                          agg_sh.at[db.at[2 * q + kk]],
                                          ssem[kk]).wait()
                    pltpu.async_copy(x2_hbm.at[gb.at[2 * qn + kk]],
                                     rows[kk], gsem[kk])

        @pl.loop(0, NPAIR // 2 - 1)
        def _(p2):
            body(2 * p2, 0, True, True)
            body(2 * p2 + 1, 1, True, True)

        body(NPAIR - 2, 0, False, True)
        body(NPAIR - 1, 1, False, False)
        for kk in range(2):
            pltpu.make_async_copy(rows[kk], agg_sh.at[db.at[2 + kk]],
                                  ssem[kk]).wait()

        plsc.subcore_barrier()

        pltpu.sync_copy(agg_sh.at[pl.ds(s * ROWS_PER_SUB, ROWS_PER_SUB)],
                        agg_hbm.at[c, pl.ds(s * ROWS_PER_SUB, ROWS_PER_SUB)])

    return sc_kernel(x2, packed)


def _sc_degrees(packed):
    @functools.partial(
        pl.kernel,
        out_type=jax.ShapeDtypeStruct((N_CORES, N_SUBCORES, DEG_PAD),
                                      jnp.float32),
        mesh=_MESH,
        compiler_params=_sc_compiler_params(),
        scratch_types=[
            pltpu.VMEM((W_CHUNKS, CHUNK), jnp.int32),
            pltpu.VMEM((DEG_PAD,), jnp.float32),
        ],
    )
    def deg_kernel(pk_hbm, deg_hbm, slab_v, deg_v):
        c = lax.axis_index("c")
        s = lax.axis_index("s")
        w = s * N_CORES + c

        pltpu.sync_copy(pk_hbm.at[pl.ds(w * W_CHUNKS, W_CHUNKS)], slab_v)

        @pl.loop(0, DEG_PAD, step=16)
        def _(i):
            deg_v[pl.ds(i, 16)] = jnp.zeros((16,), jnp.float32)

        ones16 = jnp.full((16,), 1.0, jnp.float32)

        @pl.loop(0, W_CHUNKS)
        def _(i):
            @pl.loop(0, CHUNK, step=16)
            def _(t):
                d = slab_v[i, pl.ds(t, 16)] & PK_MASK
                plsc.addupdate_scatter(deg_v, [d], ones16)

        pltpu.sync_copy(deg_v, deg_hbm.at[c, s])

    return deg_kernel(packed)


BLOCK_M = 400
GRID_M = N_NODES // BLOCK_M


def _tc_head(x, agg, deg4, W_self, W_neigh, bs2, W1, b12, W2, b22):
    def body(x_ref, agg_ref, deg_ref, ws_ref, wn_ref, bs_ref,
             w1_ref, b1_ref, w2_ref, b2_ref, o_ref):
        i = pl.program_id(0)
        dblk = deg_ref[:, :, pl.ds(i, 1), :]
        deg_sum = jnp.sum(dblk, axis=(0, 1, 2))
        inv_row = 1.0 / jnp.maximum(deg_sum, 1.0)
        inv = jnp.transpose(inv_row.reshape(1, BLOCK_M))
        a0 = agg_ref[0] * inv
        a1 = agg_ref[1] * inv
        h = (jnp.dot(x_ref[...], ws_ref[...], preferred_element_type=jnp.float32)
             + jnp.dot(a0, wn_ref[:D_HALF, :], preferred_element_type=jnp.float32)
             + jnp.dot(a1, wn_ref[D_HALF:, :], preferred_element_type=jnp.float32)
             + bs_ref[...])
        h = jnp.maximum(h, 0.0)
        h1 = jnp.maximum(
            jnp.dot(h, w1_ref[...], preferred_element_type=jnp.float32)
            + b1_ref[...], 0.0)
        o_ref[...] = (jnp.dot(h1, w2_ref[...], preferred_element_type=jnp.float32)
                      + b2_ref[...])

    return pl.pallas_call(
        body,
        grid=(GRID_M,),
        in_specs=[
            pl.BlockSpec((BLOCK_M, D_FEAT), lambda i: (i, 0)),
            pl.BlockSpec((N_CORES, BLOCK_M, D_HALF), lambda i: (0, i, 0)),
            pl.BlockSpec((N_CORES, N_SUBCORES, DEG_PAD // BLOCK_M, BLOCK_M),
                         lambda i: (0, 0, 0, 0)),
            pl.BlockSpec((D_FEAT, D_FEAT), lambda i: (0, 0)),
            pl.BlockSpec((D_FEAT, D_FEAT), lambda i: (0, 0)),
            pl.BlockSpec((1, D_FEAT), lambda i: (0, 0)),
            pl.BlockSpec((D_FEAT, D_HID), lambda i: (0, 0)),
            pl.BlockSpec((1, D_HID), lambda i: (0, 0)),
            pl.BlockSpec((D_HID, D_FEAT), lambda i: (0, 0)),
            pl.BlockSpec((1, D_FEAT), lambda i: (0, 0)),
        ],
        out_specs=pl.BlockSpec((BLOCK_M, D_FEAT), lambda i: (i, 0)),
        out_shape=jax.ShapeDtypeStruct((N_NODES, D_FEAT), jnp.float32),
        compiler_params=pltpu.CompilerParams(
            dimension_semantics=("parallel",)),
    )(x, agg, deg4, W_self, W_neigh, bs2, W1, b12, W2, b22)


def kernel(x, edge_index, W_self, W_neigh, b_sage, W1, b1, W2, b2):
    src = edge_index[0].astype(jnp.int32)
    dst = edge_index[1].astype(jnp.int32)
    pad = E_PAD - N_EDGES
    src = jnp.concatenate([src, jnp.zeros((pad,), jnp.int32)])
    dst = jnp.concatenate([dst, jnp.full((pad,), DUMMY_DST, jnp.int32)])
    packed = ((src << PK_SHIFT) | dst).reshape(E_PAD // CHUNK, CHUNK)
    x2 = x.reshape(2 * N_NODES, D_HALF)

    agg = _sc_aggregate(x2, packed)
    deg = _sc_degrees(packed)
    deg4 = deg.reshape(N_CORES, N_SUBCORES, DEG_PAD // BLOCK_M, BLOCK_M)

    return _tc_head(x, agg, deg4, W_self, W_neigh,
                    b_sage.reshape(1, D_FEAT), W1, b1.reshape(1, D_HID),
                    W2, b2.reshape(1, D_FEAT))


# speedup vs baseline: 3.7720x; 1.0061x over previous
import dataclasses
import functools

import jax
import jax.numpy as jnp
from jax import lax
from jax.experimental import pallas as pl
from jax.experimental.pallas import tpu as pltpu
from jax.experimental.pallas import tpu_sc as plsc

N_NODES = 10000
N_EDGES = 160000
D_FEAT = 256
D_HALF = 128
D_HID = 1024

N_CORES = 2
N_SUBCORES = 16

N_PAD = 10240
ROWS_PER_SUB = N_PAD // N_SUBCORES
DEG_PAD = 12000
E_PAD = 163840
E_PER_S = E_PAD // N_SUBCORES
CHUNK = 80
NCH = E_PER_S // CHUNK
NPAIR = NCH // 2
W_CHUNKS = E_PAD // (32 * CHUNK)
DUMMY_DST = N_NODES
PK_SHIFT = 14
PK_MASK = (1 << PK_SHIFT) - 1


def _sc_compiler_params():
    cp = pltpu.CompilerParams()
    if "needs_layout_passes" in pltpu.CompilerParams.__dataclass_fields__:
        cp = dataclasses.replace(cp, needs_layout_passes=False)
    return cp


_MESH = plsc.VectorSubcoreMesh(
    core_axis_name="c", subcore_axis_name="s",
    num_cores=N_CORES, num_subcores=N_SUBCORES)


def _sc_aggregate(x2, packed):
    @functools.partial(
        pl.kernel,
        out_type=jax.ShapeDtypeStruct((N_CORES, N_PAD, D_HALF), jnp.float32),
        mesh=_MESH,
        compiler_params=_sc_compiler_params(),
        scratch_types=[
            pltpu.VMEM((2, 2, CHUNK), jnp.int32),
            pltpu.VMEM((4, CHUNK), jnp.int32),
            pltpu.VMEM((4, CHUNK), jnp.int32),
            pltpu.VMEM((CHUNK, D_HALF), jnp.float32),
            pltpu.VMEM((CHUNK, D_HALF), jnp.float32),
            pltpu.VMEM_SHARED((N_PAD, D_HALF), jnp.float32),
            pltpu.SemaphoreType.DMA,
            pltpu.SemaphoreType.DMA,
            pltpu.SemaphoreType.DMA,
            pltpu.SemaphoreType.DMA,
            pltpu.SemaphoreType.DMA,
            pltpu.SemaphoreType.DMA,
        ],
    )
    def sc_kernel(x2_hbm, pk_hbm, agg_hbm,
                  pb, gb, db, rows0_v, rows1_v, agg_sh,
                  g0, g1, s0, s1, i0, i1):
        c = lax.axis_index("c")
        s = lax.axis_index("s")
        rows = (rows0_v, rows1_v)
        gsem = (g0, g1)
        ssem = (s0, s1)
        isem = (i0, i1)
        base = s * NCH

        def unpack(q):
            for kk in range(2):
                for t in range(0, CHUNK, 16):
                    v = pb[q, kk, pl.ds(t, 16)]
                    gb[2 * q + kk, pl.ds(t, 16)] = (v >> PK_SHIFT) * 2 + c
                    db[2 * q + kk, pl.ds(t, 16)] = v & PK_MASK

        @pl.loop(0, CHUNK)
        def _(i):
            @pl.loop(0, D_HALF, step=16)
            def _(j):
                rows0_v[i, pl.ds(j, 16)] = jnp.zeros((16,), jnp.float32)

        @pl.loop(0, ROWS_PER_SUB // CHUNK)
        def _(t):
            pltpu.sync_copy(
                rows0_v, agg_sh.at[pl.ds(s * ROWS_PER_SUB + t * CHUNK, CHUNK)])

        pltpu.sync_copy(pk_hbm.at[pl.ds(base, 2)], pb.at[0])
        unpack(0)
        pltpu.async_copy(pk_hbm.at[pl.ds(base + 2, 2)], pb.at[1], i1)
        pltpu.async_copy(x2_hbm.at[gb.at[0]], rows0_v, g0)
        pltpu.async_copy(x2_hbm.at[gb.at[1]], rows1_v, g1)
        plsc.subcore_barrier()

        def body(p, q, pf_idx, pf_gather):
            qn = 1 - q
            if pf_idx:
                pltpu.async_copy(pk_hbm.at[pl.ds(base + 2 * p + 4, 2)],
                                 pb.at[q], isem[q])
            for kk in range(2):
                pltpu.make_async_copy(x2_hbm.at[gb.at[2 * q + kk]],
                                      rows[kk], gsem[kk]).wait()
                pltpu.async_copy(rows[kk], agg_sh.at[db.at[2 * q + kk]],
                                 ssem[kk], add=True)
            if pf_gather:
                pltpu.make_async_copy(pk_hbm.at[pl.ds(base, 2)],
                                      pb.at[qn], isem[qn]).wait()
                unpack(qn)
                for kk in range(2):
                    pltpu.make_async_copy(rows[kk],
                                          agg_sh.at[db.at[2 * q + kk]],
                                          ssem[kk]).wait()
                    pltpu.async_copy(x2_hbm.at[gb.at[2 * qn + kk]],
                                     rows[kk], gsem[kk])

        @pl.loop(0, NPAIR // 2 - 1)
        def _(p2):
            body(2 * p2, 0, True, True)
            body(2 * p2 + 1, 1, True, True)

        body(NPAIR - 2, 0, False, True)
        body(NPAIR - 1, 1, False, False)
        for kk in range(2):
            pltpu.make_async_copy(rows[kk], agg_sh.at[db.at[2 + kk]],
                                  ssem[kk]).wait()

        plsc.subcore_barrier()

        pltpu.sync_copy(agg_sh.at[pl.ds(s * ROWS_PER_SUB, ROWS_PER_SUB)],
                        agg_hbm.at[c, pl.ds(s * ROWS_PER_SUB, ROWS_PER_SUB)])

    return sc_kernel(x2, packed)


def _sc_degrees(packed):
    @functools.partial(
        pl.kernel,
        out_type=jax.ShapeDtypeStruct((N_CORES, N_SUBCORES, DEG_PAD),
                                      jnp.float32),
        mesh=_MESH,
        compiler_params=_sc_compiler_params(),
        scratch_types=[
            pltpu.VMEM((W_CHUNKS, CHUNK), jnp.int32),
            pltpu.VMEM((DEG_PAD,), jnp.float32),
        ],
    )
    def deg_kernel(pk_hbm, deg_hbm, slab_v, deg_v):
        c = lax.axis_index("c")
        s = lax.axis_index("s")
        w = s * N_CORES + c

        pltpu.sync_copy(pk_hbm.at[pl.ds(w * W_CHUNKS, W_CHUNKS)], slab_v)

        @pl.loop(0, DEG_PAD, step=16)
        def _(i):
            deg_v[pl.ds(i, 16)] = jnp.zeros((16,), jnp.float32)

        ones16 = jnp.full((16,), 1.0, jnp.float32)

        @pl.loop(0, W_CHUNKS)
        def _(i):
            @pl.loop(0, CHUNK, step=16)
            def _(t):
                d = slab_v[i, pl.ds(t, 16)] & PK_MASK
                plsc.addupdate_scatter(deg_v, [d], ones16)

        pltpu.sync_copy(deg_v, deg_hbm.at[c, s])

    return deg_kernel(packed)


BLOCK_M = 1000
GRID_M = N_NODES // BLOCK_M


def _tc_head(x, agg, deg4, W_self, W_neigh, bs2, W1, b12, W2, b22):
    def body(x_ref, agg_ref, deg_ref, ws_ref, wn_ref, bs_ref,
             w1_ref, b1_ref, w2_ref, b2_ref, o_ref):
        i = pl.program_id(0)
        dblk = deg_ref[:, :, pl.ds(i, 1), :]
        deg_sum = jnp.sum(dblk, axis=(0, 1, 2))
        inv_row = 1.0 / jnp.maximum(deg_sum, 1.0)
        inv = jnp.transpose(inv_row.reshape(1, BLOCK_M))
        a0 = agg_ref[0] * inv
        a1 = agg_ref[1] * inv
        h = (jnp.dot(x_ref[...], ws_ref[...], preferred_element_type=jnp.float32)
             + jnp.dot(a0, wn_ref[:D_HALF, :], preferred_element_type=jnp.float32)
             + jnp.dot(a1, wn_ref[D_HALF:, :], preferred_element_type=jnp.float32)
             + bs_ref[...])
        h = jnp.maximum(h, 0.0)
        h1 = jnp.maximum(
            jnp.dot(h, w1_ref[...], preferred_element_type=jnp.float32)
            + b1_ref[...], 0.0)
        o_ref[...] = (jnp.dot(h1, w2_ref[...], preferred_element_type=jnp.float32)
                      + b2_ref[...])

    return pl.pallas_call(
        body,
        grid=(GRID_M,),
        in_specs=[
            pl.BlockSpec((BLOCK_M, D_FEAT), lambda i: (i, 0)),
            pl.BlockSpec((N_CORES, BLOCK_M, D_HALF), lambda i: (0, i, 0)),
            pl.BlockSpec((N_CORES, N_SUBCORES, DEG_PAD // BLOCK_M, BLOCK_M),
                         lambda i: (0, 0, 0, 0)),
            pl.BlockSpec((D_FEAT, D_FEAT), lambda i: (0, 0)),
            pl.BlockSpec((D_FEAT, D_FEAT), lambda i: (0, 0)),
            pl.BlockSpec((1, D_FEAT), lambda i: (0, 0)),
            pl.BlockSpec((D_FEAT, D_HID), lambda i: (0, 0)),
            pl.BlockSpec((1, D_HID), lambda i: (0, 0)),
            pl.BlockSpec((D_HID, D_FEAT), lambda i: (0, 0)),
            pl.BlockSpec((1, D_FEAT), lambda i: (0, 0)),
        ],
        out_specs=pl.BlockSpec((BLOCK_M, D_FEAT), lambda i: (i, 0)),
        out_shape=jax.ShapeDtypeStruct((N_NODES, D_FEAT), jnp.float32),
        compiler_params=pltpu.CompilerParams(
            dimension_semantics=("parallel",)),
    )(x, agg, deg4, W_self, W_neigh, bs2, W1, b12, W2, b22)


def kernel(x, edge_index, W_self, W_neigh, b_sage, W1, b1, W2, b2):
    src = edge_index[0].astype(jnp.int32)
    dst = edge_index[1].astype(jnp.int32)
    pad = E_PAD - N_EDGES
    src = jnp.concatenate([src, jnp.zeros((pad,), jnp.int32)])
    dst = jnp.concatenate([dst, jnp.full((pad,), DUMMY_DST, jnp.int32)])
    packed = ((src << PK_SHIFT) | dst).reshape(E_PAD // CHUNK, CHUNK)
    x2 = x.reshape(2 * N_NODES, D_HALF)

    agg = _sc_aggregate(x2, packed)
    deg = _sc_degrees(packed)
    deg4 = deg.reshape(N_CORES, N_SUBCORES, DEG_PAD // BLOCK_M, BLOCK_M)

    return _tc_head(x, agg, deg4, W_self, W_neigh,
                    b_sage.reshape(1, D_FEAT), W1, b1.reshape(1, D_HID),
                    W2, b2.reshape(1, D_FEAT))
